# baseline (speedup 1.0000x reference)
import functools

import jax
import jax.numpy as jnp
from jax.experimental import pallas as pl
from jax.experimental.pallas import tpu as pltpu

_C = 8
_K = 64
_STEPS = 4
_MIN_MATCH = 3
_NEG = -1e9


def _transpose_body(n_ref, nt_ref):
    nt_ref[...] = n_ref[...].T.astype(jnp.int8)


def _build_nt(neighbors, blk=512):
    n = neighbors.shape[0]
    g = n // blk
    return pl.pallas_call(
        _transpose_body,
        grid=(g, g),
        in_specs=[pl.BlockSpec((blk, blk), lambda i, j: (j, i))],
        out_specs=pl.BlockSpec((blk, blk), lambda i, j: (i, j)),
        out_shape=jax.ShapeDtypeStruct((n, n), jnp.int8),
    )(neighbors)


def _accum_body(idx_ref, w_ref, counts_ref, row_ref, out_ref):
    j = pl.program_id(0)

    @pl.when(j == 0)
    def _():
        out_ref[...] = counts_ref[...]

    c = j // _K
    n = row_ref.shape[-1]
    w = w_ref[j].astype(jnp.float32)
    row = row_ref[...].reshape(1, n).astype(jnp.float32)
    out_ref[pl.ds(c, 1), :] += w * row


def _accum_counts(nt3, counts, idx, w):
    n = nt3.shape[0]
    grid_spec = pltpu.PrefetchScalarGridSpec(
        num_scalar_prefetch=2,
        grid=(_C * _K,),
        in_specs=[
            pl.BlockSpec((_C, n), lambda j, i_ref, w_ref: (0, 0)),
            pl.BlockSpec((1, 1, n), lambda j, i_ref, w_ref: (i_ref[j], 0, 0)),
        ],
        out_specs=pl.BlockSpec((_C, n), lambda j, i_ref, w_ref: (0, 0)),
    )
    return pl.pallas_call(
        _accum_body,
        grid_spec=grid_spec,
        out_shape=jax.ShapeDtypeStruct((_C, n), jnp.float32),
    )(idx, w, counts, nt3)


def _step_body(counts_ref, sims_ref, cate_in_ref, em_in_ref,
               cate_out_ref, em_out_ref, idx_ref, w_ref, out3_ref,
               scores_ref, probs_ref, *, mm):
    counts = counts_ref[...]
    em = em_in_ref[...]
    valid = (counts > mm).astype(jnp.float32) * (1.0 - em)
    pools = (jnp.max(valid, axis=0, keepdims=True) > 0).astype(jnp.float32)
    probs = sims_ref[...] * pools
    probs_ref[...] = probs
    scores_ref[...] = jnp.where(valid > 0, probs, _NEG)
    cate_out_ref[...] = cate_in_ref[...]
    em_out_ref[...] = em

    lane = jax.lax.broadcasted_iota(jnp.int32, counts.shape, 1)

    def body(k, _):
        s = scores_ref[...]
        m = jnp.max(s, axis=1, keepdims=True)
        cand = jnp.where(s == m, lane, jnp.int32(2 ** 30))
        idxv = jnp.min(cand, axis=1, keepdims=True)
        onehot = lane == idxv
        oh_f = onehot.astype(jnp.float32)
        cate = cate_out_ref[...]
        wv = 1.0 - jnp.sum(jnp.where(onehot, cate, 0.0), axis=1,
                           keepdims=True)
        cate_out_ref[...] = jnp.maximum(cate, oh_f)
        scores_ref[...] = jnp.where(onehot, -2e9, s)
        em_out_ref[...] = jnp.maximum(em_out_ref[...],
                                      jnp.max(oh_f, axis=0, keepdims=True))
        mat = jax.lax.dot_general(
            oh_f, probs_ref[...], (((1,), (1,)), ((), ())),
            preferred_element_type=jnp.float32)
        out3_ref[pl.ds(k, 1), :, :] = mat[None]
        idx_ref[pl.ds(k, 1), :] = idxv.reshape(1, _C)
        w_ref[pl.ds(k, 1), :] = wv.reshape(1, _C)
        return 0

    jax.lax.fori_loop(0, _K, body, 0)


def _step_call(counts, simsT, cateT, em, mm):
    n = counts.shape[1]
    f = functools.partial(_step_body, mm=mm)
    return pl.pallas_call(
        f,
        in_specs=[
            pl.BlockSpec((_C, n), lambda: (0, 0)),
            pl.BlockSpec((_C, n), lambda: (0, 0)),
            pl.BlockSpec((_C, n), lambda: (0, 0)),
            pl.BlockSpec((1, n), lambda: (0, 0)),
        ],
        out_specs=[
            pl.BlockSpec((_C, n), lambda: (0, 0)),
            pl.BlockSpec((1, n), lambda: (0, 0)),
            pl.BlockSpec((_K, _C), lambda: (0, 0)),
            pl.BlockSpec((_K, _C), lambda: (0, 0)),
            pl.BlockSpec((_K, _C, _C), lambda: (0, 0, 0)),
        ],
        out_shape=[
            jax.ShapeDtypeStruct((_C, n), jnp.float32),
            jax.ShapeDtypeStruct((1, n), jnp.float32),
            jax.ShapeDtypeStruct((_K, _C), jnp.int32),
            jax.ShapeDtypeStruct((_K, _C), jnp.float32),
            jax.ShapeDtypeStruct((_K, _C, _C), jnp.float32),
        ],
        scratch_shapes=[
            pltpu.VMEM((_C, n), jnp.float32),
            pltpu.VMEM((_C, n), jnp.float32),
        ],
    )(counts, simsT, cateT, em)


def kernel(seeds, es, neighbors):
    n, d = es.shape
    seeds = seeds.astype(jnp.int32)
    seeds2 = seeds.reshape(_C, _K)

    eqm = seeds2[:, :, None] == seeds2[:, None, :]
    earlier = jnp.tril(jnp.ones((_K, _K), jnp.bool_), -1)
    dup = jnp.any(eqm & earlier[None], axis=2)
    w0 = jnp.where(dup, 0, 1).astype(jnp.int32).reshape(-1)

    em = jnp.zeros((n,), jnp.float32).at[seeds].set(1.0).reshape(1, n)
    cateT = jnp.zeros((_C, n), jnp.float32).at[
        jnp.arange(_C)[:, None], seeds2].set(1.0)

    nt3 = _build_nt(neighbors).reshape(n, 1, n)
    counts = _accum_counts(nt3, jnp.zeros((_C, n), jnp.float32), seeds, w0)

    cats = es[seeds].reshape(-1, _K, d).mean(axis=1)
    an = es / (jnp.linalg.norm(es, axis=-1, keepdims=True) + 1e-8)
    bn = cats / (jnp.linalg.norm(cats, axis=-1, keepdims=True) + 1e-8)
    simsT = ((an @ bn.T) * 0.5 + 0.5).T

    outputs, selects = [], []
    for t in range(_STEPS):
        mm = 2 if t > 2 else max(2, _MIN_MATCH - t)
        cateT, em, idxkc, wkc, out3 = _step_call(counts, simsT, cateT, em, mm)
        last = idxkc.T.reshape(-1)
        selects.append(last)
        outputs.append(jnp.transpose(out3, (1, 0, 2)).reshape(_C * _K, _C))
        if t < _STEPS - 1:
            counts = _accum_counts(nt3, counts, last,
                                   wkc.T.reshape(-1).astype(jnp.int32))
    return jnp.stack(outputs), jnp.stack(selects)


# speedup vs baseline: 1.5681x; 1.5681x over previous
import functools

import jax
import jax.numpy as jnp
from jax.experimental import pallas as pl
from jax.experimental.pallas import tpu as pltpu

_C = 8
_K = 64
_STEPS = 4
_MIN_MATCH = 3
_NEG = -1e9


def _transpose_body(n_ref, nt_ref):
    nt_ref[...] = n_ref[...].T.astype(jnp.int8)


def _build_nt(neighbors, blk=512):
    n = neighbors.shape[0]
    g = n // blk
    return pl.pallas_call(
        _transpose_body,
        grid=(g, g),
        in_specs=[pl.BlockSpec((blk, blk), lambda i, j: (j, i))],
        out_specs=pl.BlockSpec((blk, blk), lambda i, j: (i, j)),
        out_shape=jax.ShapeDtypeStruct((n, n), jnp.int8),
    )(neighbors)


_GB = 4


def _gather_body(idx_ref, r0, r1, r2, r3, out_ref):
    del idx_ref
    for m, r in enumerate((r0, r1, r2, r3)):
        out_ref[0, m, :] = r[0, 0, :]


def _delta_body(w_ref, rows_ref, counts_ref, out_ref):
    delta = jax.lax.dot_general(
        w_ref[...], rows_ref[...], (((1,), (0,)), ((), ())),
        preferred_element_type=jnp.int32)
    out_ref[...] = counts_ref[...] + delta.astype(jnp.float32)


def _accum_counts(nt3, counts, idx, wmat):
    n = nt3.shape[0]
    g = (_C * _K) // _GB
    grid_spec = pltpu.PrefetchScalarGridSpec(
        num_scalar_prefetch=1,
        grid=(g,),
        in_specs=[
            pl.BlockSpec((1, 1, n),
                         lambda j, i_ref, m=m: (i_ref[_GB * j + m], 0, 0))
            for m in range(_GB)
        ],
        out_specs=pl.BlockSpec((1, _GB, n), lambda j, i_ref: (j, 0, 0)),
    )
    rows = pl.pallas_call(
        _gather_body,
        grid_spec=grid_spec,
        out_shape=jax.ShapeDtypeStruct((g, _GB, n), jnp.int8),
    )(idx, nt3, nt3, nt3, nt3).reshape(_C * _K, n)
    return pl.pallas_call(
        _delta_body,
        in_specs=[
            pl.BlockSpec((_C, _C * _K), lambda: (0, 0)),
            pl.BlockSpec((_C * _K, n), lambda: (0, 0)),
            pl.BlockSpec((_C, n), lambda: (0, 0)),
        ],
        out_specs=pl.BlockSpec((_C, n), lambda: (0, 0)),
        out_shape=jax.ShapeDtypeStruct((_C, n), jnp.float32),
    )(wmat, rows, counts)


def _weight_mat(w):
    cls = jnp.arange(_C * _K, dtype=jnp.int32) // _K
    return ((jnp.arange(_C, dtype=jnp.int32)[:, None] == cls[None, :])
            .astype(jnp.int8) * w[None, :].astype(jnp.int8))


def _step_body(counts_ref, sims_ref, cate_in_ref, em_in_ref,
               cate_out_ref, em_out_ref, idx_ref, out3_ref,
               scores_ref, oh_ref, *, mm):
    counts = counts_ref[...]
    em = em_in_ref[...]
    valid = (counts > mm).astype(jnp.float32) * (1.0 - em)
    pools = (jnp.max(valid, axis=0, keepdims=True) > 0).astype(jnp.float32)
    probs = sims_ref[...] * pools
    scores_ref[...] = jnp.where(valid > 0, probs, _NEG)

    lane = jax.lax.broadcasted_iota(jnp.int32, counts.shape, 1)

    def body(k, _):
        s = scores_ref[...]
        m = jnp.max(s, axis=1, keepdims=True)
        cand = jnp.where(s == m, lane, jnp.int32(2 ** 30))
        idxv = jnp.min(cand, axis=1, keepdims=True)
        onehot = lane == idxv
        scores_ref[...] = jnp.where(onehot, -2e9, s)
        oh_ref[pl.ds(k, 1), :, :] = onehot.astype(jnp.float32)[None]
        idx_ref[pl.ds(k, 1), :] = idxv.reshape(1, _C)
        return 0

    jax.lax.fori_loop(0, _K, body, 0)

    sel = (scores_ref[...] == -2e9).astype(jnp.float32)
    cate_out_ref[...] = jnp.maximum(cate_in_ref[...], sel)
    em_out_ref[...] = jnp.maximum(em, jnp.max(sel, axis=0, keepdims=True))
    n = counts.shape[1]
    ohm = oh_ref[...].reshape(_K * _C, n)
    out3_ref[...] = jax.lax.dot_general(
        ohm, probs, (((1,), (1,)), ((), ())),
        preferred_element_type=jnp.float32).reshape(_K, _C, _C)


def _step_call(counts, simsT, cateT, em, mm):
    n = counts.shape[1]
    f = functools.partial(_step_body, mm=mm)
    return pl.pallas_call(
        f,
        in_specs=[
            pl.BlockSpec((_C, n), lambda: (0, 0)),
            pl.BlockSpec((_C, n), lambda: (0, 0)),
            pl.BlockSpec((_C, n), lambda: (0, 0)),
            pl.BlockSpec((1, n), lambda: (0, 0)),
        ],
        out_specs=[
            pl.BlockSpec((_C, n), lambda: (0, 0)),
            pl.BlockSpec((1, n), lambda: (0, 0)),
            pl.BlockSpec((_K, _C), lambda: (0, 0)),
            pl.BlockSpec((_K, _C, _C), lambda: (0, 0, 0)),
        ],
        out_shape=[
            jax.ShapeDtypeStruct((_C, n), jnp.float32),
            jax.ShapeDtypeStruct((1, n), jnp.float32),
            jax.ShapeDtypeStruct((_K, _C), jnp.int32),
            jax.ShapeDtypeStruct((_K, _C, _C), jnp.float32),
        ],
        scratch_shapes=[
            pltpu.VMEM((_C, n), jnp.float32),
            pltpu.VMEM((_K, _C, n), jnp.float32),
        ],
    )(counts, simsT, cateT, em)


def kernel(seeds, es, neighbors):
    n, d = es.shape
    seeds = seeds.astype(jnp.int32)
    seeds2 = seeds.reshape(_C, _K)

    eqm = seeds2[:, :, None] == seeds2[:, None, :]
    earlier = jnp.tril(jnp.ones((_K, _K), jnp.bool_), -1)
    dup = jnp.any(eqm & earlier[None], axis=2)
    w0 = jnp.where(dup, 0, 1).astype(jnp.int32).reshape(-1)

    em = jnp.zeros((n,), jnp.float32).at[seeds].set(1.0).reshape(1, n)
    cateT = jnp.zeros((_C, n), jnp.float32).at[
        jnp.arange(_C)[:, None], seeds2].set(1.0)

    nt3 = _build_nt(neighbors).reshape(n, 1, n)
    counts = _accum_counts(nt3, jnp.zeros((_C, n), jnp.float32), seeds,
                           _weight_mat(w0))

    cats = es[seeds].reshape(-1, _K, d).mean(axis=1)
    an = es / (jnp.linalg.norm(es, axis=-1, keepdims=True) + 1e-8)
    bn = cats / (jnp.linalg.norm(cats, axis=-1, keepdims=True) + 1e-8)
    simsT = ((an @ bn.T) * 0.5 + 0.5).T

    outputs, selects = [], []
    for t in range(_STEPS):
        mm = 2 if t > 2 else max(2, _MIN_MATCH - t)
        cate_prev = cateT
        cateT, em, idxkc, out3 = _step_call(counts, simsT, cateT, em, mm)
        last = idxkc.T.reshape(-1)
        selects.append(last)
        outputs.append(jnp.transpose(out3, (1, 0, 2)).reshape(_C * _K, _C))
        if t < _STEPS - 1:
            wkc = 1 - cate_prev[jnp.arange(_C)[None, :], idxkc]
            counts = _accum_counts(nt3, counts, last,
                                   _weight_mat(wkc.T.reshape(-1)
                                               .astype(jnp.int32)))
    return jnp.stack(outputs), jnp.stack(selects)


# speedup vs baseline: 1.9810x; 1.2634x over previous
import functools

import jax
import jax.numpy as jnp
from jax.experimental import pallas as pl
from jax.experimental.pallas import tpu as pltpu

_C = 8
_K = 64
_STEPS = 4
_MIN_MATCH = 3
_NEG = -1e9


def _transpose_body(n_ref, nt_ref):
    nt_ref[...] = n_ref[...].T.astype(jnp.int8)


def _build_nt(neighbors, blk=512):
    n = neighbors.shape[0]
    g = n // blk
    return pl.pallas_call(
        _transpose_body,
        grid=(g, g),
        in_specs=[pl.BlockSpec((blk, blk), lambda i, j: (j, i))],
        out_specs=pl.BlockSpec((blk, blk), lambda i, j: (i, j)),
        out_shape=jax.ShapeDtypeStruct((n, n), jnp.int8),
    )(neighbors)


_GB = 32


def _accum_body(idx_ref, w_ref, counts_ref, *refs):
    row_refs, out_ref = refs[:-1], refs[-1]
    j = pl.program_id(0)

    @pl.when(j == 0)
    def _():
        out_ref[...] = counts_ref[...]

    n = out_ref.shape[-1]
    rows = jnp.concatenate([r[...].reshape(1, n) for r in row_refs], axis=0)
    delta = jax.lax.dot_general(
        w_ref[...].reshape(_C, _GB), rows, (((1,), (0,)), ((), ())),
        preferred_element_type=jnp.int32)
    out_ref[...] += delta.astype(jnp.float32)


def _accum_counts(nt3, counts, idx, wmat):
    n = nt3.shape[0]
    g = (_C * _K) // _GB
    grid_spec = pltpu.PrefetchScalarGridSpec(
        num_scalar_prefetch=1,
        grid=(g,),
        in_specs=[
            pl.BlockSpec((1, _C, _GB), lambda j, i_ref: (j, 0, 0)),
            pl.BlockSpec((_C, n), lambda j, i_ref: (0, 0)),
        ] + [
            pl.BlockSpec((1, 1, n),
                         lambda j, i_ref, m=m: (i_ref[_GB * j + m], 0, 0))
            for m in range(_GB)
        ],
        out_specs=pl.BlockSpec((_C, n), lambda j, i_ref: (0, 0)),
    )
    return pl.pallas_call(
        _accum_body,
        grid_spec=grid_spec,
        out_shape=jax.ShapeDtypeStruct((_C, n), jnp.float32),
    )(idx, wmat, counts, *([nt3] * _GB))


def _weight_mat(w):
    cls = jnp.arange(_C * _K, dtype=jnp.int32) // _K
    wm = ((jnp.arange(_C, dtype=jnp.int32)[:, None] == cls[None, :])
          .astype(jnp.int8) * w[None, :].astype(jnp.int8))
    return wm.reshape(_C, (_C * _K) // _GB, _GB).transpose(1, 0, 2)


def _step_body(counts_ref, sims_ref, cate_in_ref, em_in_ref,
               cate_out_ref, em_out_ref, idx_ref, out3_ref,
               scores_ref, oh_ref, *, mm):
    counts = counts_ref[...]
    em = em_in_ref[...]
    valid = (counts > mm).astype(jnp.float32) * (1.0 - em)
    pools = (jnp.max(valid, axis=0, keepdims=True) > 0).astype(jnp.float32)
    probs = sims_ref[...] * pools
    scores_ref[...] = jnp.where(valid > 0, probs, _NEG)

    lane = jax.lax.broadcasted_iota(jnp.int32, counts.shape, 1)

    def body(k, _):
        s = scores_ref[...]
        m = jnp.max(s, axis=1, keepdims=True)
        cand = jnp.where(s == m, lane, jnp.int32(2 ** 30))
        idxv = jnp.min(cand, axis=1, keepdims=True)
        onehot = lane == idxv
        scores_ref[...] = jnp.where(onehot, -2e9, s)
        oh_ref[pl.ds(k, 1), :, :] = onehot.astype(jnp.float32)[None]
        idx_ref[pl.ds(k, 1), :] = idxv.reshape(1, _C)
        return 0

    jax.lax.fori_loop(0, _K, body, 0)

    sel = (scores_ref[...] == -2e9).astype(jnp.float32)
    cate_out_ref[...] = jnp.maximum(cate_in_ref[...], sel)
    em_out_ref[...] = jnp.maximum(em, jnp.max(sel, axis=0, keepdims=True))
    n = counts.shape[1]
    ohm = oh_ref[...].reshape(_K * _C, n)
    out3_ref[...] = jax.lax.dot_general(
        ohm, probs, (((1,), (1,)), ((), ())),
        preferred_element_type=jnp.float32).reshape(_K, _C, _C)


def _step_call(counts, simsT, cateT, em, mm):
    n = counts.shape[1]
    f = functools.partial(_step_body, mm=mm)
    return pl.pallas_call(
        f,
        in_specs=[
            pl.BlockSpec((_C, n), lambda: (0, 0)),
            pl.BlockSpec((_C, n), lambda: (0, 0)),
            pl.BlockSpec((_C, n), lambda: (0, 0)),
            pl.BlockSpec((1, n), lambda: (0, 0)),
        ],
        out_specs=[
            pl.BlockSpec((_C, n), lambda: (0, 0)),
            pl.BlockSpec((1, n), lambda: (0, 0)),
            pl.BlockSpec((_K, _C), lambda: (0, 0)),
            pl.BlockSpec((_K, _C, _C), lambda: (0, 0, 0)),
        ],
        out_shape=[
            jax.ShapeDtypeStruct((_C, n), jnp.float32),
            jax.ShapeDtypeStruct((1, n), jnp.float32),
            jax.ShapeDtypeStruct((_K, _C), jnp.int32),
            jax.ShapeDtypeStruct((_K, _C, _C), jnp.float32),
        ],
        scratch_shapes=[
            pltpu.VMEM((_C, n), jnp.float32),
            pltpu.VMEM((_K, _C, n), jnp.float32),
        ],
    )(counts, simsT, cateT, em)


def kernel(seeds, es, neighbors):
    n, d = es.shape
    seeds = seeds.astype(jnp.int32)
    seeds2 = seeds.reshape(_C, _K)

    eqm = seeds2[:, :, None] == seeds2[:, None, :]
    earlier = jnp.tril(jnp.ones((_K, _K), jnp.bool_), -1)
    dup = jnp.any(eqm & earlier[None], axis=2)
    w0 = jnp.where(dup, 0, 1).astype(jnp.int32).reshape(-1)

    em = jnp.zeros((n,), jnp.float32).at[seeds].set(1.0).reshape(1, n)
    cateT = jnp.zeros((_C, n), jnp.float32).at[
        jnp.arange(_C)[:, None], seeds2].set(1.0)

    nt3 = _build_nt(neighbors).reshape(n, 1, n)
    counts = _accum_counts(nt3, jnp.zeros((_C, n), jnp.float32), seeds,
                           _weight_mat(w0))

    cats = es[seeds].reshape(-1, _K, d).mean(axis=1)
    an = es / (jnp.linalg.norm(es, axis=-1, keepdims=True) + 1e-8)
    bn = cats / (jnp.linalg.norm(cats, axis=-1, keepdims=True) + 1e-8)
    simsT = ((an @ bn.T) * 0.5 + 0.5).T

    outputs, selects = [], []
    for t in range(_STEPS):
        mm = 2 if t > 2 else max(2, _MIN_MATCH - t)
        cate_prev = cateT
        cateT, em, idxkc, out3 = _step_call(counts, simsT, cateT, em, mm)
        last = idxkc.T.reshape(-1)
        selects.append(last)
        outputs.append(jnp.transpose(out3, (1, 0, 2)).reshape(_C * _K, _C))
        if t < _STEPS - 1:
            wkc = 1 - cate_prev[jnp.arange(_C)[None, :], idxkc]
            counts = _accum_counts(nt3, counts, last,
                                   _weight_mat(wkc.T.reshape(-1)
                                               .astype(jnp.int32)))
    return jnp.stack(outputs), jnp.stack(selects)


# speedup vs baseline: 2.2666x; 1.1441x over previous
import functools

import jax
import jax.numpy as jnp
from jax.experimental import pallas as pl
from jax.experimental.pallas import tpu as pltpu

_C = 8
_K = 64
_STEPS = 4
_MIN_MATCH = 3
_NEG = -1e9


def _transpose_body(n_ref, nt_ref):
    nt_ref[...] = n_ref[...].T.astype(jnp.int8)


def _build_nt(neighbors, blk=1024):
    n = neighbors.shape[0]
    g = n // blk
    return pl.pallas_call(
        _transpose_body,
        grid=(g, g),
        in_specs=[pl.BlockSpec((blk, blk), lambda i, j: (j, i))],
        out_specs=pl.BlockSpec((blk, blk), lambda i, j: (i, j)),
        out_shape=jax.ShapeDtypeStruct((n, n), jnp.int8),
    )(neighbors)


_GB = 32


def _accum_body(idx_ref, w_ref, counts_ref, *refs):
    row_refs, out_ref = refs[:-1], refs[-1]
    j = pl.program_id(0)

    @pl.when(j == 0)
    def _():
        out_ref[...] = counts_ref[...]

    n = out_ref.shape[-1]
    rows = jnp.concatenate([r[...].reshape(1, n) for r in row_refs], axis=0)
    delta = jax.lax.dot_general(
        w_ref[...].reshape(_C, _GB), rows, (((1,), (0,)), ((), ())),
        preferred_element_type=jnp.int32)
    out_ref[...] += delta.astype(jnp.float32)


def _accum_counts(nt3, counts, idx, wmat):
    n = nt3.shape[0]
    g = (_C * _K) // _GB
    grid_spec = pltpu.PrefetchScalarGridSpec(
        num_scalar_prefetch=1,
        grid=(g,),
        in_specs=[
            pl.BlockSpec((1, _C, _GB), lambda j, i_ref: (j, 0, 0)),
            pl.BlockSpec((_C, n), lambda j, i_ref: (0, 0)),
        ] + [
            pl.BlockSpec((1, 1, n),
                         lambda j, i_ref, m=m: (i_ref[_GB * j + m], 0, 0))
            for m in range(_GB)
        ],
        out_specs=pl.BlockSpec((_C, n), lambda j, i_ref: (0, 0)),
    )
    return pl.pallas_call(
        _accum_body,
        grid_spec=grid_spec,
        out_shape=jax.ShapeDtypeStruct((_C, n), jnp.float32),
    )(idx, wmat, counts, *([nt3] * _GB))


def _weight_mat(w):
    cls = jnp.arange(_C * _K, dtype=jnp.int32) // _K
    wm = ((jnp.arange(_C, dtype=jnp.int32)[:, None] == cls[None, :])
          .astype(jnp.int8) * w[None, :].astype(jnp.int8))
    return wm.reshape(_C, (_C * _K) // _GB, _GB).transpose(1, 0, 2)


def _step_body(counts_ref, sims_ref, cate_in_ref, em_in_ref,
               cate_out_ref, em_out_ref, idx_ref, out3_ref,
               scores_ref, oh_ref, *, mm):
    counts = counts_ref[...]
    em = em_in_ref[...]
    valid = (counts > mm).astype(jnp.float32) * (1.0 - em)
    pools = (jnp.max(valid, axis=0, keepdims=True) > 0).astype(jnp.float32)
    probs = sims_ref[...] * pools
    scores_ref[...] = jnp.where(valid > 0, probs, _NEG)

    lane = jax.lax.broadcasted_iota(jnp.int32, counts.shape, 1)

    def body(k, _):
        s = scores_ref[...]
        m = jnp.max(s, axis=1, keepdims=True)
        cand = jnp.where(s == m, lane, jnp.int32(2 ** 30))
        idxv = jnp.min(cand, axis=1, keepdims=True)
        onehot = lane == idxv
        scores_ref[...] = jnp.where(onehot, -2e9, s)
        oh_ref[pl.ds(k, 1), :, :] = onehot.astype(jnp.float32)[None]
        idx_ref[pl.ds(k, 1), :] = idxv.reshape(1, _C)
        return 0

    jax.lax.fori_loop(0, _K, body, 0)

    sel = (scores_ref[...] == -2e9).astype(jnp.float32)
    cate_out_ref[...] = jnp.maximum(cate_in_ref[...], sel)
    em_out_ref[...] = jnp.maximum(em, jnp.max(sel, axis=0, keepdims=True))
    n = counts.shape[1]
    ohm = oh_ref[...].reshape(_K * _C, n)
    out3_ref[...] = jax.lax.dot_general(
        ohm, probs, (((1,), (1,)), ((), ())),
        preferred_element_type=jnp.float32).reshape(_K, _C, _C)


def _step_call(counts, simsT, cateT, em, mm):
    n = counts.shape[1]
    f = functools.partial(_step_body, mm=mm)
    return pl.pallas_call(
        f,
        in_specs=[
            pl.BlockSpec((_C, n), lambda: (0, 0)),
            pl.BlockSpec((_C, n), lambda: (0, 0)),
            pl.BlockSpec((_C, n), lambda: (0, 0)),
            pl.BlockSpec((1, n), lambda: (0, 0)),
        ],
        out_specs=[
            pl.BlockSpec((_C, n), lambda: (0, 0)),
            pl.BlockSpec((1, n), lambda: (0, 0)),
            pl.BlockSpec((_K, _C), lambda: (0, 0)),
            pl.BlockSpec((_K, _C, _C), lambda: (0, 0, 0)),
        ],
        out_shape=[
            jax.ShapeDtypeStruct((_C, n), jnp.float32),
            jax.ShapeDtypeStruct((1, n), jnp.float32),
            jax.ShapeDtypeStruct((_K, _C), jnp.int32),
            jax.ShapeDtypeStruct((_K, _C, _C), jnp.float32),
        ],
        scratch_shapes=[
            pltpu.VMEM((_C, n), jnp.float32),
            pltpu.VMEM((_K, _C, n), jnp.float32),
        ],
    )(counts, simsT, cateT, em)


def kernel(seeds, es, neighbors):
    n, d = es.shape
    seeds = seeds.astype(jnp.int32)
    seeds2 = seeds.reshape(_C, _K)

    eqm = seeds2[:, :, None] == seeds2[:, None, :]
    earlier = jnp.tril(jnp.ones((_K, _K), jnp.bool_), -1)
    dup = jnp.any(eqm & earlier[None], axis=2)
    w0 = jnp.where(dup, 0, 1).astype(jnp.int32).reshape(-1)

    em = jnp.zeros((n,), jnp.float32).at[seeds].set(1.0).reshape(1, n)
    cateT = jnp.zeros((_C, n), jnp.float32).at[
        jnp.arange(_C)[:, None], seeds2].set(1.0)

    nt3 = _build_nt(neighbors).reshape(n, 1, n)
    counts = _accum_counts(nt3, jnp.zeros((_C, n), jnp.float32), seeds,
                           _weight_mat(w0))

    cats = es[seeds].reshape(-1, _K, d).mean(axis=1)
    an = es / (jnp.linalg.norm(es, axis=-1, keepdims=True) + 1e-8)
    bn = cats / (jnp.linalg.norm(cats, axis=-1, keepdims=True) + 1e-8)
    simsT = ((an @ bn.T) * 0.5 + 0.5).T

    outputs, selects = [], []
    for t in range(_STEPS):
        mm = 2 if t > 2 else max(2, _MIN_MATCH - t)
        cate_prev = cateT
        cateT, em, idxkc, out3 = _step_call(counts, simsT, cateT, em, mm)
        last = idxkc.T.reshape(-1)
        selects.append(last)
        outputs.append(jnp.transpose(out3, (1, 0, 2)).reshape(_C * _K, _C))
        if t < _STEPS - 1:
            wkc = 1 - cate_prev[jnp.arange(_C)[None, :], idxkc]
            counts = _accum_counts(nt3, counts, last,
                                   _weight_mat(wkc.T.reshape(-1)
                                               .astype(jnp.int32)))
    return jnp.stack(outputs), jnp.stack(selects)


# speedup vs baseline: 2.3399x; 1.0324x over previous
import functools

import jax
import jax.numpy as jnp
from jax.experimental import pallas as pl
from jax.experimental.pallas import tpu as pltpu

_C = 8
_K = 64
_STEPS = 4
_MIN_MATCH = 3
_NEG = -1e9


def _transpose_body(n_ref, nt_ref):
    nt_ref[...] = n_ref[...].T.astype(jnp.int8)


def _build_nt(neighbors, blk=2048):
    n = neighbors.shape[0]
    g = n // blk
    return pl.pallas_call(
        _transpose_body,
        grid=(g, g),
        in_specs=[pl.BlockSpec((blk, blk), lambda i, j: (j, i))],
        out_specs=pl.BlockSpec((blk, blk), lambda i, j: (i, j)),
        out_shape=jax.ShapeDtypeStruct((n, n), jnp.int8),
    )(neighbors)


_GB = 32


def _accum_body(idx_ref, w_ref, counts_ref, *refs):
    row_refs, out_ref = refs[:-1], refs[-1]
    j = pl.program_id(0)

    @pl.when(j == 0)
    def _():
        out_ref[...] = counts_ref[...]

    n = out_ref.shape[-1]
    rows = jnp.concatenate([r[...].reshape(1, n) for r in row_refs], axis=0)
    delta = jax.lax.dot_general(
        w_ref[...].reshape(_C, _GB), rows, (((1,), (0,)), ((), ())),
        preferred_element_type=jnp.int32)
    out_ref[...] += delta.astype(jnp.float32)


def _accum_counts(nt3, counts, idx, wmat):
    n = nt3.shape[0]
    g = (_C * _K) // _GB
    grid_spec = pltpu.PrefetchScalarGridSpec(
        num_scalar_prefetch=1,
        grid=(g,),
        in_specs=[
            pl.BlockSpec((1, _C, _GB), lambda j, i_ref: (j, 0, 0)),
            pl.BlockSpec((_C, n), lambda j, i_ref: (0, 0)),
        ] + [
            pl.BlockSpec((1, 1, n),
                         lambda j, i_ref, m=m: (i_ref[_GB * j + m], 0, 0))
            for m in range(_GB)
        ],
        out_specs=pl.BlockSpec((_C, n), lambda j, i_ref: (0, 0)),
    )
    return pl.pallas_call(
        _accum_body,
        grid_spec=grid_spec,
        out_shape=jax.ShapeDtypeStruct((_C, n), jnp.float32),
    )(idx, wmat, counts, *([nt3] * _GB))


def _weight_mat(w):
    cls = jnp.arange(_C * _K, dtype=jnp.int32) // _K
    wm = ((jnp.arange(_C, dtype=jnp.int32)[:, None] == cls[None, :])
          .astype(jnp.int8) * w[None, :].astype(jnp.int8))
    return wm.reshape(_C, (_C * _K) // _GB, _GB).transpose(1, 0, 2)


def _step_body(counts_ref, sims_ref, cate_in_ref, em_in_ref,
               cate_out_ref, em_out_ref, idx_ref, out3_ref,
               scores_ref, oh_ref, *, mm):
    counts = counts_ref[...]
    em = em_in_ref[...]
    valid = (counts > mm).astype(jnp.float32) * (1.0 - em)
    pools = (jnp.max(valid, axis=0, keepdims=True) > 0).astype(jnp.float32)
    probs = sims_ref[...] * pools
    scores_ref[...] = jnp.where(valid > 0, probs, _NEG)

    lane = jax.lax.broadcasted_iota(jnp.int32, counts.shape, 1)

    def body(k, _):
        s = scores_ref[...]
        m = jnp.max(s, axis=1, keepdims=True)
        cand = jnp.where(s == m, lane, jnp.int32(2 ** 30))
        idxv = jnp.min(cand, axis=1, keepdims=True)
        onehot = lane == idxv
        scores_ref[...] = jnp.where(onehot, -2e9, s)
        oh_ref[pl.ds(k, 1), :, :] = onehot.astype(jnp.float32)[None]
        idx_ref[pl.ds(k, 1), :] = idxv.reshape(1, _C)
        return 0

    jax.lax.fori_loop(0, _K, body, 0)

    sel = (scores_ref[...] == -2e9).astype(jnp.float32)
    cate_out_ref[...] = jnp.maximum(cate_in_ref[...], sel)
    em_out_ref[...] = jnp.maximum(em, jnp.max(sel, axis=0, keepdims=True))
    n = counts.shape[1]
    ohm = oh_ref[...].reshape(_K * _C, n)
    out3_ref[...] = jax.lax.dot_general(
        ohm, probs, (((1,), (1,)), ((), ())),
        preferred_element_type=jnp.float32).reshape(_K, _C, _C)


def _step_call(counts, simsT, cateT, em, mm):
    n = counts.shape[1]
    f = functools.partial(_step_body, mm=mm)
    return pl.pallas_call(
        f,
        in_specs=[
            pl.BlockSpec((_C, n), lambda: (0, 0)),
            pl.BlockSpec((_C, n), lambda: (0, 0)),
            pl.BlockSpec((_C, n), lambda: (0, 0)),
            pl.BlockSpec((1, n), lambda: (0, 0)),
        ],
        out_specs=[
            pl.BlockSpec((_C, n), lambda: (0, 0)),
            pl.BlockSpec((1, n), lambda: (0, 0)),
            pl.BlockSpec((_K, _C), lambda: (0, 0)),
            pl.BlockSpec((_K, _C, _C), lambda: (0, 0, 0)),
        ],
        out_shape=[
            jax.ShapeDtypeStruct((_C, n), jnp.float32),
            jax.ShapeDtypeStruct((1, n), jnp.float32),
            jax.ShapeDtypeStruct((_K, _C), jnp.int32),
            jax.ShapeDtypeStruct((_K, _C, _C), jnp.float32),
        ],
        scratch_shapes=[
            pltpu.VMEM((_C, n), jnp.float32),
            pltpu.VMEM((_K, _C, n), jnp.float32),
        ],
    )(counts, simsT, cateT, em)


def kernel(seeds, es, neighbors):
    n, d = es.shape
    seeds = seeds.astype(jnp.int32)
    seeds2 = seeds.reshape(_C, _K)

    eqm = seeds2[:, :, None] == seeds2[:, None, :]
    earlier = jnp.tril(jnp.ones((_K, _K), jnp.bool_), -1)
    dup = jnp.any(eqm & earlier[None], axis=2)
    w0 = jnp.where(dup, 0, 1).astype(jnp.int32).reshape(-1)

    em = jnp.zeros((n,), jnp.float32).at[seeds].set(1.0).reshape(1, n)
    cateT = jnp.zeros((_C, n), jnp.float32).at[
        jnp.arange(_C)[:, None], seeds2].set(1.0)

    nt3 = _build_nt(neighbors).reshape(n, 1, n)
    counts = _accum_counts(nt3, jnp.zeros((_C, n), jnp.float32), seeds,
                           _weight_mat(w0))

    cats = es[seeds].reshape(-1, _K, d).mean(axis=1)
    an = es / (jnp.linalg.norm(es, axis=-1, keepdims=True) + 1e-8)
    bn = cats / (jnp.linalg.norm(cats, axis=-1, keepdims=True) + 1e-8)
    simsT = ((an @ bn.T) * 0.5 + 0.5).T

    outputs, selects = [], []
    for t in range(_STEPS):
        mm = 2 if t > 2 else max(2, _MIN_MATCH - t)
        cate_prev = cateT
        cateT, em, idxkc, out3 = _step_call(counts, simsT, cateT, em, mm)
        last = idxkc.T.reshape(-1)
        selects.append(last)
        outputs.append(jnp.transpose(out3, (1, 0, 2)).reshape(_C * _K, _C))
        if t < _STEPS - 1:
            wkc = 1 - cate_prev[jnp.arange(_C)[None, :], idxkc]
            counts = _accum_counts(nt3, counts, last,
                                   _weight_mat(wkc.T.reshape(-1)
                                               .astype(jnp.int32)))
    return jnp.stack(outputs), jnp.stack(selects)


# speedup vs baseline: 2.3401x; 1.0001x over previous
import functools

import jax
import jax.numpy as jnp
from jax.experimental import pallas as pl
from jax.experimental.pallas import tpu as pltpu

_C = 8
_K = 64
_STEPS = 4
_MIN_MATCH = 3
_NEG = -1e9


def _transpose_body(n_ref, nt_ref):
    nt_ref[...] = n_ref[...].T.astype(jnp.int8)


def _build_nt(neighbors, br=1024, bc=4096):
    n = neighbors.shape[0]
    return pl.pallas_call(
        _transpose_body,
        grid=(n // bc, n // br),
        in_specs=[pl.BlockSpec((br, bc), lambda i, j: (j, i))],
        out_specs=pl.BlockSpec((bc, br), lambda i, j: (i, j)),
        out_shape=jax.ShapeDtypeStruct((n, n), jnp.int8),
    )(neighbors)


_GB = 32


def _accum_body(idx_ref, w_ref, counts_ref, *refs):
    row_refs, out_ref = refs[:-1], refs[-1]
    j = pl.program_id(0)

    @pl.when(j == 0)
    def _():
        out_ref[...] = counts_ref[...]

    n = out_ref.shape[-1]
    rows = jnp.concatenate([r[...].reshape(1, n) for r in row_refs], axis=0)
    delta = jax.lax.dot_general(
        w_ref[...].reshape(_C, _GB), rows, (((1,), (0,)), ((), ())),
        preferred_element_type=jnp.int32)
    out_ref[...] += delta.astype(jnp.float32)


def _accum_counts(nt3, counts, idx, wmat):
    n = nt3.shape[0]
    g = (_C * _K) // _GB
    grid_spec = pltpu.PrefetchScalarGridSpec(
        num_scalar_prefetch=1,
        grid=(g,),
        in_specs=[
            pl.BlockSpec((1, _C, _GB), lambda j, i_ref: (j, 0, 0)),
            pl.BlockSpec((_C, n), lambda j, i_ref: (0, 0)),
        ] + [
            pl.BlockSpec((1, 1, n),
                         lambda j, i_ref, m=m: (i_ref[_GB * j + m], 0, 0))
            for m in range(_GB)
        ],
        out_specs=pl.BlockSpec((_C, n), lambda j, i_ref: (0, 0)),
    )
    return pl.pallas_call(
        _accum_body,
        grid_spec=grid_spec,
        out_shape=jax.ShapeDtypeStruct((_C, n), jnp.float32),
    )(idx, wmat, counts, *([nt3] * _GB))


def _weight_mat(w):
    cls = jnp.arange(_C * _K, dtype=jnp.int32) // _K
    wm = ((jnp.arange(_C, dtype=jnp.int32)[:, None] == cls[None, :])
          .astype(jnp.int8) * w[None, :].astype(jnp.int8))
    return wm.reshape(_C, (_C * _K) // _GB, _GB).transpose(1, 0, 2)


def _step_body(counts_ref, sims_ref, cate_in_ref, em_in_ref,
               cate_out_ref, em_out_ref, idx_ref, out3_ref,
               scores_ref, oh_ref, *, mm):
    counts = counts_ref[...]
    em = em_in_ref[...]
    valid = (counts > mm).astype(jnp.float32) * (1.0 - em)
    pools = (jnp.max(valid, axis=0, keepdims=True) > 0).astype(jnp.float32)
    probs = sims_ref[...] * pools
    scores_ref[...] = jnp.where(valid > 0, probs, _NEG)

    lane = jax.lax.broadcasted_iota(jnp.int32, counts.shape, 1)

    def body(k, _):
        s = scores_ref[...]
        m = jnp.max(s, axis=1, keepdims=True)
        cand = jnp.where(s == m, lane, jnp.int32(2 ** 30))
        idxv = jnp.min(cand, axis=1, keepdims=True)
        onehot = lane == idxv
        scores_ref[...] = jnp.where(onehot, -2e9, s)
        oh_ref[pl.ds(k, 1), :, :] = onehot.astype(jnp.float32)[None]
        idx_ref[pl.ds(k, 1), :] = idxv.reshape(1, _C)
        return 0

    jax.lax.fori_loop(0, _K, body, 0)

    sel = (scores_ref[...] == -2e9).astype(jnp.float32)
    cate_out_ref[...] = jnp.maximum(cate_in_ref[...], sel)
    em_out_ref[...] = jnp.maximum(em, jnp.max(sel, axis=0, keepdims=True))
    n = counts.shape[1]
    ohm = oh_ref[...].reshape(_K * _C, n)
    out3_ref[...] = jax.lax.dot_general(
        ohm, probs, (((1,), (1,)), ((), ())),
        preferred_element_type=jnp.float32).reshape(_K, _C, _C)


def _step_call(counts, simsT, cateT, em, mm):
    n = counts.shape[1]
    f = functools.partial(_step_body, mm=mm)
    return pl.pallas_call(
        f,
        in_specs=[
            pl.BlockSpec((_C, n), lambda: (0, 0)),
            pl.BlockSpec((_C, n), lambda: (0, 0)),
            pl.BlockSpec((_C, n), lambda: (0, 0)),
            pl.BlockSpec((1, n), lambda: (0, 0)),
        ],
        out_specs=[
            pl.BlockSpec((_C, n), lambda: (0, 0)),
            pl.BlockSpec((1, n), lambda: (0, 0)),
            pl.BlockSpec((_K, _C), lambda: (0, 0)),
            pl.BlockSpec((_K, _C, _C), lambda: (0, 0, 0)),
        ],
        out_shape=[
            jax.ShapeDtypeStruct((_C, n), jnp.float32),
            jax.ShapeDtypeStruct((1, n), jnp.float32),
            jax.ShapeDtypeStruct((_K, _C), jnp.int32),
            jax.ShapeDtypeStruct((_K, _C, _C), jnp.float32),
        ],
        scratch_shapes=[
            pltpu.VMEM((_C, n), jnp.float32),
            pltpu.VMEM((_K, _C, n), jnp.float32),
        ],
    )(counts, simsT, cateT, em)


def kernel(seeds, es, neighbors):
    n, d = es.shape
    seeds = seeds.astype(jnp.int32)
    seeds2 = seeds.reshape(_C, _K)

    eqm = seeds2[:, :, None] == seeds2[:, None, :]
    earlier = jnp.tril(jnp.ones((_K, _K), jnp.bool_), -1)
    dup = jnp.any(eqm & earlier[None], axis=2)
    w0 = jnp.where(dup, 0, 1).astype(jnp.int32).reshape(-1)

    em = jnp.zeros((n,), jnp.float32).at[seeds].set(1.0).reshape(1, n)
    cateT = jnp.zeros((_C, n), jnp.float32).at[
        jnp.arange(_C)[:, None], seeds2].set(1.0)

    nt3 = _build_nt(neighbors).reshape(n, 1, n)
    counts = _accum_counts(nt3, jnp.zeros((_C, n), jnp.float32), seeds,
                           _weight_mat(w0))

    cats = es[seeds].reshape(-1, _K, d).mean(axis=1)
    an = es / (jnp.linalg.norm(es, axis=-1, keepdims=True) + 1e-8)
    bn = cats / (jnp.linalg.norm(cats, axis=-1, keepdims=True) + 1e-8)
    simsT = ((an @ bn.T) * 0.5 + 0.5).T

    outputs, selects = [], []
    for t in range(_STEPS):
        mm = 2 if t > 2 else max(2, _MIN_MATCH - t)
        cate_prev = cateT
        cateT, em, idxkc, out3 = _step_call(counts, simsT, cateT, em, mm)
        last = idxkc.T.reshape(-1)
        selects.append(last)
        outputs.append(jnp.transpose(out3, (1, 0, 2)).reshape(_C * _K, _C))
        if t < _STEPS - 1:
            wkc = 1 - cate_prev[jnp.arange(_C)[None, :], idxkc]
            counts = _accum_counts(nt3, counts, last,
                                   _weight_mat(wkc.T.reshape(-1)
                                               .astype(jnp.int32)))
    return jnp.stack(outputs), jnp.stack(selects)


# speedup vs baseline: 2.3526x; 1.0053x over previous
import functools

import jax
import jax.numpy as jnp
from jax.experimental import pallas as pl
from jax.experimental.pallas import tpu as pltpu

_C = 8
_K = 64
_STEPS = 4
_MIN_MATCH = 3
_NEG = -1e9


def _transpose_body(n_ref, nt_ref):
    nt_ref[...] = n_ref[...].T.astype(jnp.int8)


def _build_nt(neighbors, br=1024, bc=4096):
    n = neighbors.shape[0]
    return pl.pallas_call(
        _transpose_body,
        grid=(n // bc, n // br),
        in_specs=[pl.BlockSpec((br, bc), lambda i, j: (j, i))],
        out_specs=pl.BlockSpec((bc, br), lambda i, j: (i, j)),
        out_shape=jax.ShapeDtypeStruct((n, n), jnp.int8),
    )(neighbors)


_GB = 32


def _accum_body(idx_ref, w_ref, counts_ref, *refs):
    row_refs, out_ref = refs[:-1], refs[-1]
    j = pl.program_id(0)

    @pl.when(j == 0)
    def _():
        out_ref[...] = counts_ref[...]

    n = out_ref.shape[-1]
    rows = jnp.concatenate([r[...].reshape(1, n) for r in row_refs], axis=0)
    delta = jax.lax.dot_general(
        w_ref[...].reshape(_C, _GB), rows, (((1,), (0,)), ((), ())),
        preferred_element_type=jnp.int32)
    out_ref[...] += delta.astype(jnp.float32)


def _accum_counts(nt3, counts, idx, wmat):
    n = nt3.shape[0]
    g = (_C * _K) // _GB
    grid_spec = pltpu.PrefetchScalarGridSpec(
        num_scalar_prefetch=1,
        grid=(g,),
        in_specs=[
            pl.BlockSpec((1, _C, _GB), lambda j, i_ref: (j, 0, 0)),
            pl.BlockSpec((_C, n), lambda j, i_ref: (0, 0)),
        ] + [
            pl.BlockSpec((1, 1, n),
                         lambda j, i_ref, m=m: (i_ref[_GB * j + m], 0, 0))
            for m in range(_GB)
        ],
        out_specs=pl.BlockSpec((_C, n), lambda j, i_ref: (0, 0)),
    )
    return pl.pallas_call(
        _accum_body,
        grid_spec=grid_spec,
        out_shape=jax.ShapeDtypeStruct((_C, n), jnp.float32),
    )(idx, wmat, counts, *([nt3] * _GB))


def _weight_mat(w):
    cls = jnp.arange(_C * _K, dtype=jnp.int32) // _K
    wm = ((jnp.arange(_C, dtype=jnp.int32)[:, None] == cls[None, :])
          .astype(jnp.int8) * w[None, :].astype(jnp.int8))
    return wm.reshape(_C, (_C * _K) // _GB, _GB).transpose(1, 0, 2)


def _step_body(counts_ref, sims_ref, cate_in_ref, em_in_ref,
               cate_out_ref, em_out_ref, idx_ref, out3_ref,
               scores_ref, oh_ref, *, mm):
    counts = counts_ref[...]
    em = em_in_ref[...]
    valid = (counts > mm).astype(jnp.float32) * (1.0 - em)
    pools = (jnp.max(valid, axis=0, keepdims=True) > 0).astype(jnp.float32)
    probs = sims_ref[...] * pools
    scores_ref[...] = jnp.where(valid > 0, probs, _NEG)

    lane = jax.lax.broadcasted_iota(jnp.int32, counts.shape, 1)

    def body(k, _):
        s = scores_ref[...]
        m = jnp.max(s, axis=1, keepdims=True)
        cand = jnp.where(s == m, lane, jnp.int32(2 ** 30))
        idxv = jnp.min(cand, axis=1, keepdims=True)
        onehot = lane == idxv
        scores_ref[...] = jnp.where(onehot, -2e9, s)
        oh_ref[pl.ds(k, 1), :, :] = onehot.astype(jnp.float32)[None]
        idx_ref[pl.ds(k, 1), :] = idxv.reshape(1, _C)
        return 0

    jax.lax.fori_loop(0, _K, body, 0, unroll=4)

    sel = (scores_ref[...] == -2e9).astype(jnp.float32)
    cate_out_ref[...] = jnp.maximum(cate_in_ref[...], sel)
    em_out_ref[...] = jnp.maximum(em, jnp.max(sel, axis=0, keepdims=True))
    n = counts.shape[1]
    ohm = oh_ref[...].reshape(_K * _C, n)
    out3_ref[...] = jax.lax.dot_general(
        ohm, probs, (((1,), (1,)), ((), ())),
        preferred_element_type=jnp.float32).reshape(_K, _C, _C)


def _step_call(counts, simsT, cateT, em, mm):
    n = counts.shape[1]
    f = functools.partial(_step_body, mm=mm)
    return pl.pallas_call(
        f,
        in_specs=[
            pl.BlockSpec((_C, n), lambda: (0, 0)),
            pl.BlockSpec((_C, n), lambda: (0, 0)),
            pl.BlockSpec((_C, n), lambda: (0, 0)),
            pl.BlockSpec((1, n), lambda: (0, 0)),
        ],
        out_specs=[
            pl.BlockSpec((_C, n), lambda: (0, 0)),
            pl.BlockSpec((1, n), lambda: (0, 0)),
            pl.BlockSpec((_K, _C), lambda: (0, 0)),
            pl.BlockSpec((_K, _C, _C), lambda: (0, 0, 0)),
        ],
        out_shape=[
            jax.ShapeDtypeStruct((_C, n), jnp.float32),
            jax.ShapeDtypeStruct((1, n), jnp.float32),
            jax.ShapeDtypeStruct((_K, _C), jnp.int32),
            jax.ShapeDtypeStruct((_K, _C, _C), jnp.float32),
        ],
        scratch_shapes=[
            pltpu.VMEM((_C, n), jnp.float32),
            pltpu.VMEM((_K, _C, n), jnp.float32),
        ],
    )(counts, simsT, cateT, em)


def kernel(seeds, es, neighbors):
    n, d = es.shape
    seeds = seeds.astype(jnp.int32)
    seeds2 = seeds.reshape(_C, _K)

    eqm = seeds2[:, :, None] == seeds2[:, None, :]
    earlier = jnp.tril(jnp.ones((_K, _K), jnp.bool_), -1)
    dup = jnp.any(eqm & earlier[None], axis=2)
    w0 = jnp.where(dup, 0, 1).astype(jnp.int32).reshape(-1)

    em = jnp.zeros((n,), jnp.float32).at[seeds].set(1.0).reshape(1, n)
    cateT = jnp.zeros((_C, n), jnp.float32).at[
        jnp.arange(_C)[:, None], seeds2].set(1.0)

    nt3 = _build_nt(neighbors).reshape(n, 1, n)
    counts = _accum_counts(nt3, jnp.zeros((_C, n), jnp.float32), seeds,
                           _weight_mat(w0))

    cats = es[seeds].reshape(-1, _K, d).mean(axis=1)
    an = es / (jnp.linalg.norm(es, axis=-1, keepdims=True) + 1e-8)
    bn = cats / (jnp.linalg.norm(cats, axis=-1, keepdims=True) + 1e-8)
    simsT = ((an @ bn.T) * 0.5 + 0.5).T

    outputs, selects = [], []
    for t in range(_STEPS):
        mm = 2 if t > 2 else max(2, _MIN_MATCH - t)
        cate_prev = cateT
        cateT, em, idxkc, out3 = _step_call(counts, simsT, cateT, em, mm)
        last = idxkc.T.reshape(-1)
        selects.append(last)
        outputs.append(jnp.transpose(out3, (1, 0, 2)).reshape(_C * _K, _C))
        if t < _STEPS - 1:
            wkc = 1 - cate_prev[jnp.arange(_C)[None, :], idxkc]
            counts = _accum_counts(nt3, counts, last,
                                   _weight_mat(wkc.T.reshape(-1)
                                               .astype(jnp.int32)))
    return jnp.stack(outputs), jnp.stack(selects)


# speedup vs baseline: 2.3638x; 1.0048x over previous
import functools

import jax
import jax.numpy as jnp
from jax.experimental import pallas as pl
from jax.experimental.pallas import tpu as pltpu

_C = 8
_K = 64
_STEPS = 4
_MIN_MATCH = 3
_NEG = -1e9


def _transpose_body(n_ref, nt_ref):
    nt_ref[...] = n_ref[...].T.astype(jnp.int8)


def _build_nt(neighbors, br=1024, bc=4096):
    n = neighbors.shape[0]
    return pl.pallas_call(
        _transpose_body,
        grid=(n // bc, n // br),
        in_specs=[pl.BlockSpec((br, bc), lambda i, j: (j, i))],
        out_specs=pl.BlockSpec((bc, br), lambda i, j: (i, j)),
        out_shape=jax.ShapeDtypeStruct((n, n), jnp.int8),
    )(neighbors)


_GB = 32


def _accum_body(idx_ref, w_ref, counts_ref, *refs):
    row_refs, out_ref = refs[:-1], refs[-1]
    j = pl.program_id(0)

    @pl.when(j == 0)
    def _():
        out_ref[...] = counts_ref[...]

    n = out_ref.shape[-1]
    rows = jnp.concatenate([r[...].reshape(1, n) for r in row_refs], axis=0)
    delta = jax.lax.dot_general(
        w_ref[...].reshape(_C, _GB), rows, (((1,), (0,)), ((), ())),
        preferred_element_type=jnp.int32)
    out_ref[...] += delta.astype(jnp.float32)


def _accum_counts(nt3, counts, idx, wmat):
    n = nt3.shape[0]
    g = (_C * _K) // _GB
    grid_spec = pltpu.PrefetchScalarGridSpec(
        num_scalar_prefetch=1,
        grid=(g,),
        in_specs=[
            pl.BlockSpec((1, _C, _GB), lambda j, i_ref: (j, 0, 0)),
            pl.BlockSpec((_C, n), lambda j, i_ref: (0, 0)),
        ] + [
            pl.BlockSpec((1, 1, n),
                         lambda j, i_ref, m=m: (i_ref[_GB * j + m], 0, 0))
            for m in range(_GB)
        ],
        out_specs=pl.BlockSpec((_C, n), lambda j, i_ref: (0, 0)),
    )
    return pl.pallas_call(
        _accum_body,
        grid_spec=grid_spec,
        out_shape=jax.ShapeDtypeStruct((_C, n), jnp.float32),
    )(idx, wmat, counts, *([nt3] * _GB))


def _weight_mat(w):
    cls = jnp.arange(_C * _K, dtype=jnp.int32) // _K
    wm = ((jnp.arange(_C, dtype=jnp.int32)[:, None] == cls[None, :])
          .astype(jnp.int8) * w[None, :].astype(jnp.int8))
    return wm.reshape(_C, (_C * _K) // _GB, _GB).transpose(1, 0, 2)


def _step_body(counts_ref, sims_ref, cate_in_ref, em_in_ref,
               cate_out_ref, em_out_ref, idx_ref, out3_ref,
               scores_ref, oh_ref, *, mm):
    counts = counts_ref[...]
    em = em_in_ref[...]
    valid = (counts > mm).astype(jnp.float32) * (1.0 - em)
    pools = (jnp.max(valid, axis=0, keepdims=True) > 0).astype(jnp.float32)
    probs = sims_ref[...] * pools
    scores_ref[...] = jnp.where(valid > 0, probs, _NEG)

    lane = jax.lax.broadcasted_iota(jnp.int32, counts.shape, 1)

    def body(k, _):
        s = scores_ref[...]
        m = jnp.max(s, axis=1, keepdims=True)
        cand = jnp.where(s == m, lane, jnp.int32(2 ** 30))
        idxv = jnp.min(cand, axis=1, keepdims=True)
        onehot = lane == idxv
        scores_ref[...] = jnp.where(onehot, -2e9, s)
        oh_ref[pl.ds(k, 1), :, :] = onehot.astype(jnp.float32)[None]
        idx_ref[pl.ds(k, 1), :] = idxv.reshape(1, _C)
        return 0

    jax.lax.fori_loop(0, _K, body, 0, unroll=8)

    sel = (scores_ref[...] == -2e9).astype(jnp.float32)
    cate_out_ref[...] = jnp.maximum(cate_in_ref[...], sel)
    em_out_ref[...] = jnp.maximum(em, jnp.max(sel, axis=0, keepdims=True))
    n = counts.shape[1]
    ohm = oh_ref[...].reshape(_K * _C, n)
    out3_ref[...] = jax.lax.dot_general(
        ohm, probs, (((1,), (1,)), ((), ())),
        preferred_element_type=jnp.float32).reshape(_K, _C, _C)


def _step_call(counts, simsT, cateT, em, mm):
    n = counts.shape[1]
    f = functools.partial(_step_body, mm=mm)
    return pl.pallas_call(
        f,
        in_specs=[
            pl.BlockSpec((_C, n), lambda: (0, 0)),
            pl.BlockSpec((_C, n), lambda: (0, 0)),
            pl.BlockSpec((_C, n), lambda: (0, 0)),
            pl.BlockSpec((1, n), lambda: (0, 0)),
        ],
        out_specs=[
            pl.BlockSpec((_C, n), lambda: (0, 0)),
            pl.BlockSpec((1, n), lambda: (0, 0)),
            pl.BlockSpec((_K, _C), lambda: (0, 0)),
            pl.BlockSpec((_K, _C, _C), lambda: (0, 0, 0)),
        ],
        out_shape=[
            jax.ShapeDtypeStruct((_C, n), jnp.float32),
            jax.ShapeDtypeStruct((1, n), jnp.float32),
            jax.ShapeDtypeStruct((_K, _C), jnp.int32),
            jax.ShapeDtypeStruct((_K, _C, _C), jnp.float32),
        ],
        scratch_shapes=[
            pltpu.VMEM((_C, n), jnp.float32),
            pltpu.VMEM((_K, _C, n), jnp.float32),
        ],
    )(counts, simsT, cateT, em)


def kernel(seeds, es, neighbors):
    n, d = es.shape
    seeds = seeds.astype(jnp.int32)
    seeds2 = seeds.reshape(_C, _K)

    eqm = seeds2[:, :, None] == seeds2[:, None, :]
    earlier = jnp.tril(jnp.ones((_K, _K), jnp.bool_), -1)
    dup = jnp.any(eqm & earlier[None], axis=2)
    w0 = jnp.where(dup, 0, 1).astype(jnp.int32).reshape(-1)

    em = jnp.zeros((n,), jnp.float32).at[seeds].set(1.0).reshape(1, n)
    cateT = jnp.zeros((_C, n), jnp.float32).at[
        jnp.arange(_C)[:, None], seeds2].set(1.0)

    nt3 = _build_nt(neighbors).reshape(n, 1, n)
    counts = _accum_counts(nt3, jnp.zeros((_C, n), jnp.float32), seeds,
                           _weight_mat(w0))

    cats = es[seeds].reshape(-1, _K, d).mean(axis=1)
    an = es / (jnp.linalg.norm(es, axis=-1, keepdims=True) + 1e-8)
    bn = cats / (jnp.linalg.norm(cats, axis=-1, keepdims=True) + 1e-8)
    simsT = ((an @ bn.T) * 0.5 + 0.5).T

    outputs, selects = [], []
    for t in range(_STEPS):
        mm = 2 if t > 2 else max(2, _MIN_MATCH - t)
        cate_prev = cateT
        cateT, em, idxkc, out3 = _step_call(counts, simsT, cateT, em, mm)
        last = idxkc.T.reshape(-1)
        selects.append(last)
        outputs.append(jnp.transpose(out3, (1, 0, 2)).reshape(_C * _K, _C))
        if t < _STEPS - 1:
            wkc = 1 - cate_prev[jnp.arange(_C)[None, :], idxkc]
            counts = _accum_counts(nt3, counts, last,
                                   _weight_mat(wkc.T.reshape(-1)
                                               .astype(jnp.int32)))
    return jnp.stack(outputs), jnp.stack(selects)
